# baseline (speedup 1.0000x reference)
import jax
import jax.numpy as jnp
from jax.experimental import pallas as pl
from jax.experimental.pallas import tpu as pltpu

_SPEECH_ID = 31999
_ST_ID = 31995
_IGNORE = -100


def _merge_kernel(lens_ref, ids_ref, mask_ref, labels_ref,
                  feature_ref, w_ref, b_ref, prompt_ref, table_ref,
                  out_ref, omask_ref, olab_ref,
                  sp_scratch, zero_ref, sem):
    L = ids_ref.shape[-1]
    S = sp_scratch.shape[0]
    d = sp_scratch.shape[1]
    L_out = L - 1 + S
    b = pl.program_id(0)

    sp_scratch[...] = jnp.dot(feature_ref[0], w_ref[...],
                              preferred_element_type=jnp.float32) + b_ref[...]
    zero_ref[...] = jnp.zeros_like(zero_ref)

    T = jax.lax.fori_loop(0, L, lambda i, t: t + mask_ref[0, 0, i],
                          jnp.int32(0))

    def pos_body(i, carry):
        found, p = carry
        hit = jnp.logical_and(ids_ref[0, 0, i] == _SPEECH_ID,
                              jnp.logical_not(found))
        return jnp.logical_or(found, hit), jnp.where(hit, i, p)

    _, pos = jax.lax.fori_loop(0, L, pos_body,
                               (jnp.bool_(False), jnp.int32(0)))

    sl = lens_ref[b]
    total = T - 1 + sl
    ofs = L_out - total

    idx = jax.lax.broadcasted_iota(jnp.int32, (1, 1, L_out), 2)
    omask_ref[...] = idx >= ofs

    def row_body(j, carry):
        k = j - ofs
        in_pad = k < 0
        in_speech = jnp.logical_and(k >= pos, k < pos + sl)
        text_idx = jnp.where(k < pos, k, k - sl + 1)
        text_idx = jnp.clip(text_idx, 0, L - 1)
        token = ids_ref[0, 0, text_idx]
        is_prompt = jnp.logical_and(token >= _ST_ID, token <= _ST_ID + 3)
        is_text = jnp.logical_not(jnp.logical_or(in_pad, in_speech))

        olab_ref[0, 0, j] = jnp.where(is_text, labels_ref[0, 0, text_idx],
                                      jnp.int32(_IGNORE))

        dst = out_ref.at[b, j]

        @pl.when(in_pad)
        def _():
            cp = pltpu.make_async_copy(zero_ref.at[0], dst, sem)
            cp.start()
            cp.wait()

        @pl.when(in_speech)
        def _():
            sp_row = jnp.clip(k - pos, 0, S - 1)
            cp = pltpu.make_async_copy(sp_scratch.at[sp_row], dst, sem)
            cp.start()
            cp.wait()

        @pl.when(jnp.logical_and(is_text, is_prompt))
        def _():
            cp = pltpu.make_async_copy(prompt_ref.at[token - _ST_ID], dst, sem)
            cp.start()
            cp.wait()

        @pl.when(jnp.logical_and(is_text, jnp.logical_not(is_prompt)))
        def _():
            cp = pltpu.make_async_copy(table_ref.at[token], dst, sem)
            cp.start()
            cp.wait()

        return carry

    jax.lax.fori_loop(0, L_out, row_body, jnp.int32(0))


def kernel(feature, feature_lens, input_ids, attention_mask, labels,
           embed_table, prompt_embedding, W_proj, b_proj):
    B, S, d_enc = feature.shape
    L = input_ids.shape[1]
    d = embed_table.shape[1]
    L_out = L - 1 + S

    ids3 = input_ids.reshape(B, 1, L)
    mask3 = attention_mask.astype(jnp.int32).reshape(B, 1, L)
    labels3 = labels.reshape(B, 1, L)
    b2 = b_proj.reshape(1, d)

    merged, omask, olab = pl.pallas_call(
        _merge_kernel,
        grid=(B,),
        in_specs=[
            pl.BlockSpec(memory_space=pltpu.SMEM),
            pl.BlockSpec((1, 1, L), lambda b: (b, 0, 0),
                         memory_space=pltpu.SMEM),
            pl.BlockSpec((1, 1, L), lambda b: (b, 0, 0),
                         memory_space=pltpu.SMEM),
            pl.BlockSpec((1, 1, L), lambda b: (b, 0, 0),
                         memory_space=pltpu.SMEM),
            pl.BlockSpec((1, S, d_enc), lambda b: (b, 0, 0)),
            pl.BlockSpec((d_enc, d), lambda b: (0, 0)),
            pl.BlockSpec((1, d), lambda b: (0, 0)),
            pl.BlockSpec((4, d), lambda b: (0, 0)),
            pl.BlockSpec(memory_space=pl.ANY),
        ],
        out_specs=[
            pl.BlockSpec(memory_space=pl.ANY),
            pl.BlockSpec((1, 1, L_out), lambda b: (b, 0, 0)),
            pl.BlockSpec((1, 1, L_out), lambda b: (b, 0, 0),
                         memory_space=pltpu.SMEM),
        ],
        out_shape=[
            jax.ShapeDtypeStruct((B, L_out, d), jnp.float32),
            jax.ShapeDtypeStruct((B, 1, L_out), jnp.bool_),
            jax.ShapeDtypeStruct((B, 1, L_out), jnp.int32),
        ],
        scratch_shapes=[
            pltpu.VMEM((S, d), jnp.float32),
            pltpu.VMEM((8, d), jnp.float32),
            pltpu.SemaphoreType.DMA,
        ],
        compiler_params=pltpu.CompilerParams(
            dimension_semantics=("arbitrary",),
        ),
    )(feature_lens, ids3, mask3, labels3, feature, W_proj, b2,
      prompt_embedding, embed_table)
    return merged, omask.reshape(B, L_out), olab.reshape(B, L_out)


# speedup vs baseline: 9.0288x; 9.0288x over previous
import jax
import jax.numpy as jnp
from jax.experimental import pallas as pl
from jax.experimental.pallas import tpu as pltpu

_SPEECH_ID = 31999
_ST_ID = 31995
_IGNORE = -100


def _merge_kernel(lens_ref, ids_ref, mask_ref, labels_ref,
                  feature_ref, w_ref, b_ref, prompt_ref, table_ref,
                  out_ref, omask_ref, olab_ref,
                  sp_scratch, zero_ref, sem):
    L = ids_ref.shape[-1]
    S = sp_scratch.shape[0]
    d = sp_scratch.shape[1]
    L_out = L - 1 + S
    b = pl.program_id(0)

    sp_scratch[...] = jnp.dot(feature_ref[0], w_ref[...],
                              preferred_element_type=jnp.float32) + b_ref[...]
    zero_ref[...] = jnp.zeros_like(zero_ref)

    T = jax.lax.fori_loop(0, L, lambda i, t: t + mask_ref[0, 0, i],
                          jnp.int32(0))

    def pos_body(i, carry):
        found, p = carry
        hit = jnp.logical_and(ids_ref[0, 0, i] == _SPEECH_ID,
                              jnp.logical_not(found))
        return jnp.logical_or(found, hit), jnp.where(hit, i, p)

    _, pos = jax.lax.fori_loop(0, L, pos_body,
                               (jnp.bool_(False), jnp.int32(0)))

    sl = lens_ref[b]
    total = T - 1 + sl
    ofs = L_out - total

    idx = jax.lax.broadcasted_iota(jnp.int32, (1, 1, L_out), 2)
    omask_ref[...] = idx >= ofs

    NSEM = 16

    def row_body(j, carry):
        k = j - ofs
        in_pad = k < 0
        in_speech = jnp.logical_and(k >= pos, k < pos + sl)
        text_idx = jnp.where(k < pos, k, k - sl + 1)
        text_idx = jnp.clip(text_idx, 0, L - 1)
        token = ids_ref[0, 0, text_idx]
        is_prompt = jnp.logical_and(token >= _ST_ID, token <= _ST_ID + 3)
        is_text = jnp.logical_not(jnp.logical_or(in_pad, in_speech))

        olab_ref[0, 0, j] = jnp.where(is_text, labels_ref[0, 0, text_idx],
                                      jnp.int32(_IGNORE))

        slot = jax.lax.rem(j, NSEM)
        dst = out_ref.at[b, j]

        @pl.when(j >= NSEM)
        def _():
            pltpu.make_async_copy(zero_ref.at[0], zero_ref.at[1],
                                  sem.at[slot]).wait()

        @pl.when(in_pad)
        def _():
            pltpu.make_async_copy(zero_ref.at[0], dst, sem.at[slot]).start()

        @pl.when(in_speech)
        def _():
            sp_row = jnp.clip(k - pos, 0, S - 1)
            pltpu.make_async_copy(sp_scratch.at[sp_row], dst,
                                  sem.at[slot]).start()

        @pl.when(jnp.logical_and(is_text, is_prompt))
        def _():
            pltpu.make_async_copy(prompt_ref.at[token - _ST_ID], dst,
                                  sem.at[slot]).start()

        @pl.when(jnp.logical_and(is_text, jnp.logical_not(is_prompt)))
        def _():
            pltpu.make_async_copy(table_ref.at[token], dst,
                                  sem.at[slot]).start()

        return carry

    jax.lax.fori_loop(0, L_out, row_body, jnp.int32(0))

    def drain_body(i, carry):
        slot = jax.lax.rem(jnp.int32(L_out - NSEM) + i, NSEM)
        pltpu.make_async_copy(zero_ref.at[0], zero_ref.at[1],
                              sem.at[slot]).wait()
        return carry

    jax.lax.fori_loop(0, NSEM, drain_body, jnp.int32(0))


def kernel(feature, feature_lens, input_ids, attention_mask, labels,
           embed_table, prompt_embedding, W_proj, b_proj):
    B, S, d_enc = feature.shape
    L = input_ids.shape[1]
    d = embed_table.shape[1]
    L_out = L - 1 + S

    ids3 = input_ids.reshape(B, 1, L)
    mask3 = attention_mask.astype(jnp.int32).reshape(B, 1, L)
    labels3 = labels.reshape(B, 1, L)
    b2 = b_proj.reshape(1, d)

    merged, omask, olab = pl.pallas_call(
        _merge_kernel,
        grid=(B,),
        in_specs=[
            pl.BlockSpec(memory_space=pltpu.SMEM),
            pl.BlockSpec((1, 1, L), lambda b: (b, 0, 0),
                         memory_space=pltpu.SMEM),
            pl.BlockSpec((1, 1, L), lambda b: (b, 0, 0),
                         memory_space=pltpu.SMEM),
            pl.BlockSpec((1, 1, L), lambda b: (b, 0, 0),
                         memory_space=pltpu.SMEM),
            pl.BlockSpec((1, S, d_enc), lambda b: (b, 0, 0)),
            pl.BlockSpec((d_enc, d), lambda b: (0, 0)),
            pl.BlockSpec((1, d), lambda b: (0, 0)),
            pl.BlockSpec((4, d), lambda b: (0, 0)),
            pl.BlockSpec(memory_space=pl.ANY),
        ],
        out_specs=[
            pl.BlockSpec(memory_space=pl.ANY),
            pl.BlockSpec((1, 1, L_out), lambda b: (b, 0, 0)),
            pl.BlockSpec((1, 1, L_out), lambda b: (b, 0, 0),
                         memory_space=pltpu.SMEM),
        ],
        out_shape=[
            jax.ShapeDtypeStruct((B, L_out, d), jnp.float32),
            jax.ShapeDtypeStruct((B, 1, L_out), jnp.bool_),
            jax.ShapeDtypeStruct((B, 1, L_out), jnp.int32),
        ],
        scratch_shapes=[
            pltpu.VMEM((S, d), jnp.float32),
            pltpu.VMEM((8, d), jnp.float32),
            pltpu.SemaphoreType.DMA((16,)),
        ],
        compiler_params=pltpu.CompilerParams(
            dimension_semantics=("arbitrary",),
        ),
    )(feature_lens, ids3, mask3, labels3, feature, W_proj, b2,
      prompt_embedding, embed_table)
    return merged, omask.reshape(B, L_out), olab.reshape(B, L_out)
